# baseline (speedup 1.0000x reference)
import jax
import jax.numpy as jnp
from jax import lax
from jax.experimental import pallas as pl
from jax.experimental.pallas import tpu as pltpu
from jax.experimental.pallas import tpu_sc as plsc

N_NODES = 10000
D = 128
HID = 64
N_SEG = 10000
E = 320000

DH = 64
TW = 80
NC, NS = 2, 16
NW = NC * NS
CHUNK = 80
E_PER_SUB = E // NS
CHUNKS_PER_SUB = E_PER_SUB // CHUNK
NBUF = 4
ROUNDS = CHUNKS_PER_SUB // NBUF
TAIL = CHUNKS_PER_SUB - ROUNDS * NBUF
ROWS_PER_SUB = N_SEG // NS
WB = 125
DROW = 640
E_BETA = E // NW
BCHUNKS = E_BETA // CHUNK


def _prep_body(x_ref, w1_ref, b1_ref, w2_ref, b2_ref, y0_ref, y1_ref, ew_ref):
    x = x_ref[...]
    z = jnp.dot(x, w1_ref[...], preferred_element_type=jnp.float32) + b1_ref[...]
    h = jnp.where(z >= 0, z, 0.01 * z)
    w = jnp.dot(h, w2_ref[...], preferred_element_type=jnp.float32) + b2_ref[...]
    ew = jnp.exp(w)
    blk = x.shape[0]
    pad = jnp.zeros((blk, TW - DH - 1), jnp.float32)
    y0_ref[...] = jnp.concatenate([x[:, :DH] * ew, ew, pad], axis=1)
    y1_ref[...] = jnp.concatenate([x[:, DH:] * ew, ew, pad], axis=1)
    ew_ref[...] = ew


def _prep(X, W1, b1, W2, b2):
    blk = 1000
    return pl.pallas_call(
        _prep_body,
        grid=(N_NODES // blk,),
        in_specs=[
            pl.BlockSpec((blk, D), lambda i: (i, 0)),
            pl.BlockSpec((D, HID), lambda i: (0, 0)),
            pl.BlockSpec((1, HID), lambda i: (0, 0)),
            pl.BlockSpec((HID, 1), lambda i: (0, 0)),
            pl.BlockSpec((1, 1), lambda i: (0, 0)),
        ],
        out_specs=[
            pl.BlockSpec((blk, TW), lambda i: (i, 0)),
            pl.BlockSpec((blk, TW), lambda i: (i, 0)),
            pl.BlockSpec((blk, 1), lambda i: (i, 0)),
        ],
        out_shape=[
            jax.ShapeDtypeStruct((N_NODES, TW), jnp.float32),
            jax.ShapeDtypeStruct((N_NODES, TW), jnp.float32),
            jax.ShapeDtypeStruct((N_NODES, 1), jnp.float32),
        ],
    )(X, W1, b1.reshape(1, HID), W2, b2.reshape(1, 1))


def _accum_body(y0_hbm, y1_hbm, nidx_hbm, sidx_hbm, zeros_hbm,
                ud_hbm, den_hbm,
                nidx_v, sidx_v, rows_v, blk_v, den_v, ud_sp,
                gs0, gs1, gs2, gs3, ss0, ss1, ss2, ss3, wsem):
    c = lax.axis_index("c")
    s = lax.axis_index("s")
    gsems = (gs0, gs1, gs2, gs3)
    ssems = (ss0, ss1, ss2, ss3)

    pltpu.async_copy(nidx_hbm.at[pl.ds(s * CHUNKS_PER_SUB, CHUNKS_PER_SUB)],
                     nidx_v, gs0)
    pltpu.async_copy(sidx_hbm.at[pl.ds(s * CHUNKS_PER_SUB, CHUNKS_PER_SUB)],
                     sidx_v, gs1)
    row0 = s * ROWS_PER_SUB
    for k in range(ROWS_PER_SUB // WB):
        pltpu.sync_copy(zeros_hbm, ud_sp.at[pl.ds(row0 + k * WB, WB)])
    pltpu.make_async_copy(
        nidx_hbm.at[pl.ds(s * CHUNKS_PER_SUB, CHUNKS_PER_SUB)], nidx_v,
        gs0).wait()
    pltpu.make_async_copy(
        sidx_hbm.at[pl.ds(s * CHUNKS_PER_SUB, CHUNKS_PER_SUB)], sidx_v,
        gs1).wait()
    plsc.subcore_barrier()

    def run(y_hbm):
        def gstart(b, g):
            pltpu.async_copy(y_hbm.at[nidx_v.at[g]], rows_v.at[b], gsems[b])

        def gwait(b, g):
            pltpu.make_async_copy(y_hbm.at[nidx_v.at[g]], rows_v.at[b],
                                  gsems[b]).wait()

        def sstart(b, g):
            pltpu.async_copy(rows_v.at[b], ud_sp.at[sidx_v.at[g]], ssems[b],
                             add=True)

        def swait(b, g):
            pltpu.make_async_copy(rows_v.at[b], ud_sp.at[sidx_v.at[g]],
                                  ssems[b]).wait()

        for b in range(NBUF):
            gstart(b, b)

        def round_(r, _):
            for b in range(NBUF):
                g = r * NBUF + b
                gwait(b, g)
                sstart(b, g)
            for b in range(NBUF):
                g = r * NBUF + b
                swait(b, g)

                @pl.when(g + NBUF < CHUNKS_PER_SUB)
                def _():
                    gstart(b, g + NBUF)
            return 0

        lax.fori_loop(0, ROUNDS, round_, 0)
        for b in range(TAIL):
            g = ROUNDS * NBUF + b
            gwait(b, g)
            sstart(b, g)
        for b in range(TAIL):
            swait(b, ROUNDS * NBUF + b)

    @pl.when(c == 0)
    def _():
        run(y0_hbm)

    @pl.when(c == 1)
    def _():
        run(y1_hbm)

    plsc.subcore_barrier()

    lanes = lax.iota(jnp.int32, 16)
    for cc in range(NC):
        @pl.when(c == cc)
        def _():
            for k in range(ROWS_PER_SUB // WB):
                r = row0 + k * WB
                pltpu.sync_copy(ud_sp.at[pl.ds(r, WB)], blk_v)
                pltpu.async_copy(blk_v, ud_hbm.at[cc, pl.ds(r, WB)], wsem)
                for j in range(8):
                    rows16 = lanes + j * 16
                    dvals = plsc.load_gather(blk_v, [rows16, lanes * 0 + DH],
                                             mask=rows16 < WB)
                    den_v[pl.ds(k * WB + j * 16, 16)] = dvals
                pltpu.make_async_copy(blk_v, ud_hbm.at[cc, pl.ds(r, WB)],
                                      wsem).wait()

    @pl.when(c == 0)
    def _():
        pltpu.sync_copy(den_v, den_hbm.at[s])


def _accum(y0, y1, nidx2d, sidx2d, zeros):
    mesh = plsc.VectorSubcoreMesh(core_axis_name="c", subcore_axis_name="s")
    return pl.kernel(
        _accum_body,
        out_type=[
            jax.ShapeDtypeStruct((NC, N_SEG, TW), jnp.float32),
            jax.ShapeDtypeStruct((NS, DROW), jnp.float32),
        ],
        mesh=mesh,
        scratch_types=[
            pltpu.VMEM((CHUNKS_PER_SUB, CHUNK), jnp.int32),
            pltpu.VMEM((CHUNKS_PER_SUB, CHUNK), jnp.int32),
            pltpu.VMEM((NBUF, CHUNK, TW), jnp.float32),
            pltpu.VMEM((WB, TW), jnp.float32),
            pltpu.VMEM((DROW,), jnp.float32),
            pltpu.VMEM_SHARED((N_SEG, TW), jnp.float32),
        ] + [pltpu.SemaphoreType.DMA] * (2 * NBUF + 1),
        compiler_params=pltpu.CompilerParams(use_tc_tiling_on_sc=False,
                                             needs_layout_passes=False),
    )(y0, y1, nidx2d, sidx2d, zeros)


def _beta_body(ew_hbm, den_hbm, nidx_hbm, sidx_hbm, beta_hbm,
               ew_v, dtab_v, ni_v, si_v, beta_v):
    c = lax.axis_index("c")
    s = lax.axis_index("s")
    wid = c * NS + s
    base = wid * E_BETA

    pltpu.sync_copy(ew_hbm, ew_v)
    pltpu.sync_copy(den_hbm, dtab_v)
    pltpu.sync_copy(nidx_hbm.at[pl.ds(base, E_BETA)], ni_v)
    pltpu.sync_copy(sidx_hbm.at[pl.ds(base, E_BETA)], si_v)

    def bstep(i, _):
        off = i * 16
        ni = ni_v[pl.ds(off, 16)]
        si = si_v[pl.ds(off, 16)]
        e = plsc.load_gather(ew_v, [ni])
        d = plsc.load_gather(dtab_v, [si // ROWS_PER_SUB,
                                      si % ROWS_PER_SUB])
        beta_v[pl.ds(off, 16)] = e / d
        return 0

    lax.fori_loop(0, E_BETA // 16, bstep, 0)
    pltpu.sync_copy(beta_v, beta_hbm.at[pl.ds(base, E_BETA)])


def _beta(ew1d, den, node_index, segment_ids):
    mesh = plsc.VectorSubcoreMesh(core_axis_name="c", subcore_axis_name="s")
    return pl.kernel(
        _beta_body,
        out_type=jax.ShapeDtypeStruct((E,), jnp.float32),
        mesh=mesh,
        scratch_types=[
            pltpu.VMEM((N_NODES,), jnp.float32),
            pltpu.VMEM((NS, DROW), jnp.float32),
            pltpu.VMEM((E_BETA,), jnp.int32),
            pltpu.VMEM((E_BETA,), jnp.int32),
            pltpu.VMEM((E_BETA,), jnp.float32),
        ],
        compiler_params=pltpu.CompilerParams(use_tc_tiling_on_sc=False,
                                             needs_layout_passes=False),
    )(ew1d, den, node_index, segment_ids)


def _final_body(ud0_ref, ud1_ref, z_ref):
    u0 = ud0_ref[...]
    u1 = ud1_ref[...]
    d = u0[:, DH:DH + 1]
    rcp = jnp.where(d > 0, 1.0 / d, 0.0)
    zp = jnp.concatenate([u0[:, :DH], u1[:, :DH]], axis=1) * rcp
    z_ref[...] = jnp.tanh(jnp.where(zp >= 0, zp, 0.01 * zp))


def _final(ud0, ud1):
    blk = 1000
    return pl.pallas_call(
        _final_body,
        grid=(N_SEG // blk,),
        in_specs=[
            pl.BlockSpec((blk, TW), lambda i: (i, 0)),
            pl.BlockSpec((blk, TW), lambda i: (i, 0)),
        ],
        out_specs=pl.BlockSpec((blk, D), lambda i: (i, 0)),
        out_shape=jax.ShapeDtypeStruct((N_SEG, D), jnp.float32),
    )(ud0, ud1)


@jax.jit
def kernel(X, node_index, segment_ids, W1, b1, W2, b2):
    y0, y1, ew = _prep(X, W1, b1, W2, b2)
    nidx2d = node_index.reshape(E // CHUNK, CHUNK)
    sidx2d = segment_ids.reshape(E // CHUNK, CHUNK)
    zeros = jnp.zeros((WB, TW), jnp.float32)
    ud, den = _accum(y0, y1, nidx2d, sidx2d, zeros)
    beta = _beta(ew.reshape(N_NODES), den, node_index, segment_ids)
    z = _final(ud[0], ud[1])
    return z, beta


# speedup vs baseline: 1.0280x; 1.0280x over previous
import jax
import jax.numpy as jnp
from jax import lax
from jax.experimental import pallas as pl
from jax.experimental.pallas import tpu as pltpu
from jax.experimental.pallas import tpu_sc as plsc

N_NODES = 10000
D = 128
HID = 64
N_SEG = 10000
E = 320000

DH = 64
TW = 80
NC, NS = 2, 16
NW = NC * NS
CHUNK = 80
E_PER_SUB = E // NS
CHUNKS_PER_SUB = E_PER_SUB // CHUNK
NBUF = 4
ROUNDS = CHUNKS_PER_SUB // NBUF
TAIL = CHUNKS_PER_SUB - ROUNDS * NBUF
ROWS_PER_SUB = N_SEG // NS
WB = 125
DROW = 640
E_BETA = E // NW
BCHUNKS = E_BETA // CHUNK


def _prep_body(x_ref, w1_ref, b1_ref, w2_ref, b2_ref, y0_ref, y1_ref, ew_ref):
    x = x_ref[...]
    z = jnp.dot(x, w1_ref[...], preferred_element_type=jnp.float32) + b1_ref[...]
    h = jnp.where(z >= 0, z, 0.01 * z)
    w = jnp.dot(h, w2_ref[...], preferred_element_type=jnp.float32) + b2_ref[...]
    ew = jnp.exp(w)
    blk = x.shape[0]
    pad = jnp.zeros((blk, TW - DH - 1), jnp.float32)
    y0_ref[...] = jnp.concatenate([x[:, :DH] * ew, ew, pad], axis=1)
    y1_ref[...] = jnp.concatenate([x[:, DH:] * ew, ew, pad], axis=1)
    ew_ref[...] = ew


def _prep(X, W1, b1, W2, b2):
    blk = 1000
    return pl.pallas_call(
        _prep_body,
        grid=(N_NODES // blk,),
        in_specs=[
            pl.BlockSpec((blk, D), lambda i: (i, 0)),
            pl.BlockSpec((D, HID), lambda i: (0, 0)),
            pl.BlockSpec((1, HID), lambda i: (0, 0)),
            pl.BlockSpec((HID, 1), lambda i: (0, 0)),
            pl.BlockSpec((1, 1), lambda i: (0, 0)),
        ],
        out_specs=[
            pl.BlockSpec((blk, TW), lambda i: (i, 0)),
            pl.BlockSpec((blk, TW), lambda i: (i, 0)),
            pl.BlockSpec((blk, 1), lambda i: (i, 0)),
        ],
        out_shape=[
            jax.ShapeDtypeStruct((N_NODES, TW), jnp.float32),
            jax.ShapeDtypeStruct((N_NODES, TW), jnp.float32),
            jax.ShapeDtypeStruct((N_NODES, 1), jnp.float32),
        ],
    )(X, W1, b1.reshape(1, HID), W2, b2.reshape(1, 1))


def _accum_body(y0_hbm, y1_hbm, nidx_hbm, sidx_hbm, zeros_hbm,
                ud_hbm, den_hbm,
                nidx_v, sidx_v, rows_v, blk_v, den_v, ud_sp,
                gs0, gs1, gs2, gs3, ss0, ss1, ss2, ss3, wsem):
    c = lax.axis_index("c")
    s = lax.axis_index("s")
    gsems = (gs0, gs1, gs2, gs3)
    ssems = (ss0, ss1, ss2, ss3)

    pltpu.sync_copy(zeros_hbm, blk_v)
    row0 = s * ROWS_PER_SUB
    for k in range(ROWS_PER_SUB // WB):
        pltpu.sync_copy(blk_v, ud_sp.at[pl.ds(row0 + k * WB, WB)])

    pltpu.sync_copy(nidx_hbm.at[pl.ds(s * CHUNKS_PER_SUB, CHUNKS_PER_SUB)],
                    nidx_v)
    pltpu.sync_copy(sidx_hbm.at[pl.ds(s * CHUNKS_PER_SUB, CHUNKS_PER_SUB)],
                    sidx_v)
    plsc.subcore_barrier()

    def run(y_hbm):
        def gstart(b, g):
            pltpu.async_copy(y_hbm.at[nidx_v.at[g]], rows_v.at[b], gsems[b])

        def gwait(b, g):
            pltpu.make_async_copy(y_hbm.at[nidx_v.at[g]], rows_v.at[b],
                                  gsems[b]).wait()

        def sstart(b, g):
            pltpu.async_copy(rows_v.at[b], ud_sp.at[sidx_v.at[g]], ssems[b],
                             add=True)

        def swait(b, g):
            pltpu.make_async_copy(rows_v.at[b], ud_sp.at[sidx_v.at[g]],
                                  ssems[b]).wait()

        for b in range(NBUF):
            gstart(b, b)

        def round_(r, _):
            for b in range(NBUF):
                g = r * NBUF + b
                gwait(b, g)
                sstart(b, g)
            for b in range(NBUF):
                g = r * NBUF + b
                swait(b, g)

                @pl.when(g + NBUF < CHUNKS_PER_SUB)
                def _():
                    gstart(b, g + NBUF)
            return 0

        lax.fori_loop(0, ROUNDS, round_, 0)
        for b in range(TAIL):
            g = ROUNDS * NBUF + b
            gwait(b, g)
            sstart(b, g)
        for b in range(TAIL):
            swait(b, ROUNDS * NBUF + b)

    @pl.when(c == 0)
    def _():
        run(y0_hbm)

    @pl.when(c == 1)
    def _():
        run(y1_hbm)

    plsc.subcore_barrier()

    lanes = lax.iota(jnp.int32, 16)
    for cc in range(NC):
        @pl.when(c == cc)
        def _():
            for k in range(ROWS_PER_SUB // WB):
                r = row0 + k * WB
                pltpu.sync_copy(ud_sp.at[pl.ds(r, WB)], blk_v)
                pltpu.async_copy(blk_v, ud_hbm.at[cc, pl.ds(r, WB)], wsem)
                for j in range(8):
                    rows16 = lanes + j * 16
                    dvals = plsc.load_gather(blk_v, [rows16, lanes * 0 + DH],
                                             mask=rows16 < WB)
                    den_v[pl.ds(k * WB + j * 16, 16)] = dvals
                pltpu.make_async_copy(blk_v, ud_hbm.at[cc, pl.ds(r, WB)],
                                      wsem).wait()

    @pl.when(c == 0)
    def _():
        pltpu.sync_copy(den_v, den_hbm.at[s])


def _accum(y0, y1, nidx2d, sidx2d, zeros):
    mesh = plsc.VectorSubcoreMesh(core_axis_name="c", subcore_axis_name="s")
    return pl.kernel(
        _accum_body,
        out_type=[
            jax.ShapeDtypeStruct((NC, N_SEG, TW), jnp.float32),
            jax.ShapeDtypeStruct((NS, DROW), jnp.float32),
        ],
        mesh=mesh,
        scratch_types=[
            pltpu.VMEM((CHUNKS_PER_SUB, CHUNK), jnp.int32),
            pltpu.VMEM((CHUNKS_PER_SUB, CHUNK), jnp.int32),
            pltpu.VMEM((NBUF, CHUNK, TW), jnp.float32),
            pltpu.VMEM((WB, TW), jnp.float32),
            pltpu.VMEM((DROW,), jnp.float32),
            pltpu.VMEM_SHARED((N_SEG, TW), jnp.float32),
        ] + [pltpu.SemaphoreType.DMA] * (2 * NBUF + 1),
        compiler_params=pltpu.CompilerParams(use_tc_tiling_on_sc=False,
                                             needs_layout_passes=False),
    )(y0, y1, nidx2d, sidx2d, zeros)


def _beta_body(ew_hbm, den_hbm, nidx_hbm, sidx_hbm, beta_hbm,
               ew_v, dtab_v, ni_v, si_v, beta_v):
    c = lax.axis_index("c")
    s = lax.axis_index("s")
    wid = c * NS + s
    base = wid * E_BETA

    pltpu.sync_copy(ew_hbm, ew_v)
    pltpu.sync_copy(den_hbm, dtab_v)
    pltpu.sync_copy(nidx_hbm.at[pl.ds(base, E_BETA)], ni_v)
    pltpu.sync_copy(sidx_hbm.at[pl.ds(base, E_BETA)], si_v)

    def bstep(i, _):
        off = i * 16
        ni = ni_v[pl.ds(off, 16)]
        si = si_v[pl.ds(off, 16)]
        e = plsc.load_gather(ew_v, [ni])
        d = plsc.load_gather(dtab_v, [si // ROWS_PER_SUB,
                                      si % ROWS_PER_SUB])
        beta_v[pl.ds(off, 16)] = e / d
        return 0

    lax.fori_loop(0, E_BETA // 16, bstep, 0)
    pltpu.sync_copy(beta_v, beta_hbm.at[pl.ds(base, E_BETA)])


def _beta(ew1d, den, node_index, segment_ids):
    mesh = plsc.VectorSubcoreMesh(core_axis_name="c", subcore_axis_name="s")
    return pl.kernel(
        _beta_body,
        out_type=jax.ShapeDtypeStruct((E,), jnp.float32),
        mesh=mesh,
        scratch_types=[
            pltpu.VMEM((N_NODES,), jnp.float32),
            pltpu.VMEM((NS, DROW), jnp.float32),
            pltpu.VMEM((E_BETA,), jnp.int32),
            pltpu.VMEM((E_BETA,), jnp.int32),
            pltpu.VMEM((E_BETA,), jnp.float32),
        ],
        compiler_params=pltpu.CompilerParams(use_tc_tiling_on_sc=False,
                                             needs_layout_passes=False),
    )(ew1d, den, node_index, segment_ids)


def _final_body(ud0_ref, ud1_ref, z_ref):
    u0 = ud0_ref[...]
    u1 = ud1_ref[...]
    d = u0[:, DH:DH + 1]
    rcp = jnp.where(d > 0, 1.0 / d, 0.0)
    zp = jnp.concatenate([u0[:, :DH], u1[:, :DH]], axis=1) * rcp
    z_ref[...] = jnp.tanh(jnp.where(zp >= 0, zp, 0.01 * zp))


def _final(ud0, ud1):
    blk = 1000
    return pl.pallas_call(
        _final_body,
        grid=(N_SEG // blk,),
        in_specs=[
            pl.BlockSpec((blk, TW), lambda i: (i, 0)),
            pl.BlockSpec((blk, TW), lambda i: (i, 0)),
        ],
        out_specs=pl.BlockSpec((blk, D), lambda i: (i, 0)),
        out_shape=jax.ShapeDtypeStruct((N_SEG, D), jnp.float32),
    )(ud0, ud1)


@jax.jit
def kernel(X, node_index, segment_ids, W1, b1, W2, b2):
    y0, y1, ew = _prep(X, W1, b1, W2, b2)
    nidx2d = node_index.reshape(E // CHUNK, CHUNK)
    sidx2d = segment_ids.reshape(E // CHUNK, CHUNK)
    zeros = jnp.zeros((WB, TW), jnp.float32)
    ud, den = _accum(y0, y1, nidx2d, sidx2d, zeros)
    beta = _beta(ew.reshape(N_NODES), den, node_index, segment_ids)
    z = _final(ud[0], ud[1])
    return z, beta


# speedup vs baseline: 1.0289x; 1.0009x over previous
import jax
import jax.numpy as jnp
from jax import lax
from jax.experimental import pallas as pl
from jax.experimental.pallas import tpu as pltpu
from jax.experimental.pallas import tpu_sc as plsc

N_NODES = 10000
D = 128
HID = 64
N_SEG = 10000
E = 320000

DH = 64
TW = 80
NC, NS = 2, 16
NW = NC * NS
CHUNK = 80
E_PER_SUB = E // NS
CHUNKS_PER_SUB = E_PER_SUB // CHUNK
NBUF = 4
ROUNDS = CHUNKS_PER_SUB // NBUF
TAIL = CHUNKS_PER_SUB - ROUNDS * NBUF
ROWS_PER_SUB = N_SEG // NS
WB = 125
DROW = 640
E_BETA = E // NW


def _prep_body(x_ref, w1_ref, b1_ref, w2_ref, b2_ref, y0_ref, y1_ref, ew_ref):
    x = x_ref[...]
    z = jnp.dot(x, w1_ref[...], preferred_element_type=jnp.float32) + b1_ref[...]
    h = jnp.where(z >= 0, z, 0.01 * z)
    w = jnp.dot(h, w2_ref[...], preferred_element_type=jnp.float32) + b2_ref[...]
    ew = jnp.exp(w)
    blk = x.shape[0]
    pad = jnp.zeros((blk, TW - DH - 1), jnp.float32)
    y0_ref[...] = jnp.concatenate([x[:, :DH] * ew, ew, pad], axis=1)
    y1_ref[...] = jnp.concatenate([x[:, DH:] * ew, ew, pad], axis=1)
    ew_ref[...] = ew


def _prep(X, W1, b1, W2, b2):
    blk = 1000
    return pl.pallas_call(
        _prep_body,
        grid=(N_NODES // blk,),
        in_specs=[
            pl.BlockSpec((blk, D), lambda i: (i, 0)),
            pl.BlockSpec((D, HID), lambda i: (0, 0)),
            pl.BlockSpec((1, HID), lambda i: (0, 0)),
            pl.BlockSpec((HID, 1), lambda i: (0, 0)),
            pl.BlockSpec((1, 1), lambda i: (0, 0)),
        ],
        out_specs=[
            pl.BlockSpec((blk, TW), lambda i: (i, 0)),
            pl.BlockSpec((blk, TW), lambda i: (i, 0)),
            pl.BlockSpec((blk, 1), lambda i: (i, 0)),
        ],
        out_shape=[
            jax.ShapeDtypeStruct((N_NODES, TW), jnp.float32),
            jax.ShapeDtypeStruct((N_NODES, TW), jnp.float32),
            jax.ShapeDtypeStruct((N_NODES, 1), jnp.float32),
        ],
    )(X, W1, b1.reshape(1, HID), W2, b2.reshape(1, 1))


def _accum_body(y0_hbm, y1_hbm, nidx_hbm, sidx_hbm, zeros_hbm,
                ud_hbm, den_hbm,
                nidx_v, sidx_v, rows_v, blk_v, den_v, ud_sp,
                gs0, gs1, gs2, gs3, ss0, ss1, ss2, ss3, wsem):
    c = lax.axis_index("c")
    s = lax.axis_index("s")
    gsems = (gs0, gs1, gs2, gs3)
    ssems = (ss0, ss1, ss2, ss3)

    pltpu.sync_copy(zeros_hbm, blk_v)
    row0 = s * ROWS_PER_SUB
    for k in range(ROWS_PER_SUB // WB):
        pltpu.sync_copy(blk_v, ud_sp.at[pl.ds(row0 + k * WB, WB)])

    pltpu.sync_copy(nidx_hbm.at[pl.ds(s * CHUNKS_PER_SUB, CHUNKS_PER_SUB)],
                    nidx_v)
    pltpu.sync_copy(sidx_hbm.at[pl.ds(s * CHUNKS_PER_SUB, CHUNKS_PER_SUB)],
                    sidx_v)
    plsc.subcore_barrier()

    def run(y_hbm):
        def gstart(b, g):
            pltpu.async_copy(y_hbm.at[nidx_v.at[g]], rows_v.at[b], gsems[b])

        def gwait(b, g):
            pltpu.make_async_copy(y_hbm.at[nidx_v.at[g]], rows_v.at[b],
                                  gsems[b]).wait()

        def sstart(b, g):
            pltpu.async_copy(rows_v.at[b], ud_sp.at[sidx_v.at[g]], ssems[b],
                             add=True)

        def swait(b, g):
            pltpu.make_async_copy(rows_v.at[b], ud_sp.at[sidx_v.at[g]],
                                  ssems[b]).wait()

        for b in range(NBUF):
            gstart(b, b)

        def round_(r, _):
            for b in range(NBUF):
                g = r * NBUF + b
                gwait(b, g)
                sstart(b, g)
            for b in range(NBUF):
                g = r * NBUF + b
                swait(b, g)

                @pl.when(g + NBUF < CHUNKS_PER_SUB)
                def _():
                    gstart(b, g + NBUF)
            return 0

        lax.fori_loop(0, ROUNDS, round_, 0)
        for b in range(TAIL):
            g = ROUNDS * NBUF + b
            gwait(b, g)
            sstart(b, g)
        for b in range(TAIL):
            swait(b, ROUNDS * NBUF + b)

    @pl.when(c == 0)
    def _():
        run(y0_hbm)

    @pl.when(c == 1)
    def _():
        run(y1_hbm)

    plsc.subcore_barrier()

    lanes = lax.iota(jnp.int32, 16)
    for cc in range(NC):
        @pl.when(c == cc)
        def _():
            for k in range(ROWS_PER_SUB // WB):
                r = row0 + k * WB
                pltpu.sync_copy(ud_sp.at[pl.ds(r, WB)], blk_v)
                pltpu.async_copy(blk_v, ud_hbm.at[cc, pl.ds(r, WB)], wsem)
                for j in range(8):
                    rows16 = lanes + j * 16
                    dvals = plsc.load_gather(blk_v, [rows16, lanes * 0 + DH],
                                             mask=rows16 < WB)
                    den_v[pl.ds(k * WB + j * 16, 16)] = dvals
                pltpu.make_async_copy(blk_v, ud_hbm.at[cc, pl.ds(r, WB)],
                                      wsem).wait()

    @pl.when(c == 0)
    def _():
        pltpu.sync_copy(den_v, den_hbm.at[s])


def _accum(y0, y1, nidx2d, sidx2d, zeros):
    mesh = plsc.VectorSubcoreMesh(core_axis_name="c", subcore_axis_name="s")
    return pl.kernel(
        _accum_body,
        out_type=[
            jax.ShapeDtypeStruct((NC, N_SEG, TW), jnp.float32),
            jax.ShapeDtypeStruct((NS, DROW), jnp.float32),
        ],
        mesh=mesh,
        scratch_types=[
            pltpu.VMEM((CHUNKS_PER_SUB, CHUNK), jnp.int32),
            pltpu.VMEM((CHUNKS_PER_SUB, CHUNK), jnp.int32),
            pltpu.VMEM((NBUF, CHUNK, TW), jnp.float32),
            pltpu.VMEM((WB, TW), jnp.float32),
            pltpu.VMEM((DROW,), jnp.float32),
            pltpu.VMEM_SHARED((N_SEG, TW), jnp.float32),
        ] + [pltpu.SemaphoreType.DMA] * (2 * NBUF + 1),
        compiler_params=pltpu.CompilerParams(use_tc_tiling_on_sc=False,
                                             needs_layout_passes=False),
    )(y0, y1, nidx2d, sidx2d, zeros)


def _beta_body(ew_hbm, den_hbm, nidx_hbm, sidx_hbm, beta_hbm,
               ew_v, dtab_v, ni_v, si_v, beta_v):
    c = lax.axis_index("c")
    s = lax.axis_index("s")
    wid = c * NS + s
    base = wid * E_BETA

    pltpu.sync_copy(ew_hbm, ew_v)
    pltpu.sync_copy(den_hbm, dtab_v)
    pltpu.sync_copy(nidx_hbm.at[pl.ds(base, E_BETA)], ni_v)
    pltpu.sync_copy(sidx_hbm.at[pl.ds(base, E_BETA)], si_v)

    def bstep(i, _):
        off = i * 16
        ni = ni_v[pl.ds(off, 16)]
        si = si_v[pl.ds(off, 16)]
        e = plsc.load_gather(ew_v, [ni])
        d = plsc.load_gather(dtab_v, [si // ROWS_PER_SUB,
                                      si % ROWS_PER_SUB])
        beta_v[pl.ds(off, 16)] = e / d
        return 0

    lax.fori_loop(0, E_BETA // 16, bstep, 0)
    pltpu.sync_copy(beta_v, beta_hbm.at[pl.ds(base, E_BETA)])


def _beta(ew1d, den, node_index, segment_ids):
    mesh = plsc.VectorSubcoreMesh(core_axis_name="c", subcore_axis_name="s")
    return pl.kernel(
        _beta_body,
        out_type=jax.ShapeDtypeStruct((E,), jnp.float32),
        mesh=mesh,
        scratch_types=[
            pltpu.VMEM((N_NODES,), jnp.float32),
            pltpu.VMEM((NS, DROW), jnp.float32),
            pltpu.VMEM((E_BETA,), jnp.int32),
            pltpu.VMEM((E_BETA,), jnp.int32),
            pltpu.VMEM((E_BETA,), jnp.float32),
        ],
        compiler_params=pltpu.CompilerParams(use_tc_tiling_on_sc=False,
                                             needs_layout_passes=False),
    )(ew1d, den, node_index, segment_ids)


def _final_body(ud0_ref, ud1_ref, z_ref):
    u0 = ud0_ref[...]
    u1 = ud1_ref[...]
    d = u0[:, DH:DH + 1]
    rcp = jnp.where(d > 0, 1.0 / d, 0.0)
    zp = jnp.concatenate([u0[:, :DH], u1[:, :DH]], axis=1) * rcp
    z_ref[...] = jnp.tanh(jnp.where(zp >= 0, zp, 0.01 * zp))


def _final(ud0, ud1):
    blk = 1000
    return pl.pallas_call(
        _final_body,
        grid=(N_SEG // blk,),
        in_specs=[
            pl.BlockSpec((blk, TW), lambda i: (i, 0)),
            pl.BlockSpec((blk, TW), lambda i: (i, 0)),
        ],
        out_specs=pl.BlockSpec((blk, D), lambda i: (i, 0)),
        out_shape=jax.ShapeDtypeStruct((N_SEG, D), jnp.float32),
    )(ud0, ud1)


@jax.jit
def kernel(X, node_index, segment_ids, W1, b1, W2, b2):
    y0, y1, ew = _prep(X, W1, b1, W2, b2)
    nidx2d = node_index.reshape(E // CHUNK, CHUNK)
    sidx2d = segment_ids.reshape(E // CHUNK, CHUNK)
    zeros = jnp.zeros((WB, TW), jnp.float32)
    ud, den = _accum(y0, y1, nidx2d, sidx2d, zeros)
    beta = _beta(ew.reshape(N_NODES), den, node_index, segment_ids)
    z = _final(ud[0], ud[1])
    return z, beta
